# baseline (speedup 1.0000x reference)
import functools

import jax
import jax.numpy as jnp
from jax import lax
from jax.experimental import pallas as pl
from jax.experimental.pallas import tpu as pltpu
from jax.experimental.pallas import tpu_sc as plsc

B = 4
S = 2048
DIM = 768
COND_SIZE = 6

NC = 2
NS = 16
NW = NC * NS
SROWS = S // NW
CHUNK = 32
NCHUNK = SROWS // CHUNK
LANES = 16
NGROUPS = DIM // LANES


def _affine_body(cid_ref, cond_emb_ref, wh_ref, wg_ref, wb_ref, clg_ref,
                 clb_ref, gamma_ref, beta_ref):
  conds6 = lax.dot_general(cond_emb_ref[...], wh_ref[...],
                           (((1,), (1,)), ((), ())),
                           preferred_element_type=jnp.float32)
  gamma6 = clg_ref[...] + lax.dot_general(conds6, wg_ref[...],
                                          (((1,), (1,)), ((), ())),
                                          preferred_element_type=jnp.float32)
  beta6 = clb_ref[...] + lax.dot_general(conds6, wb_ref[...],
                                         (((1,), (1,)), ((), ())),
                                         preferred_element_type=jnp.float32)
  onehot = (cid_ref[...] == lax.broadcasted_iota(jnp.int32, (B, COND_SIZE), 1)
            ).astype(jnp.float32)
  gamma_ref[...] = lax.dot_general(onehot, gamma6, (((1,), (0,)), ((), ())),
                                   preferred_element_type=jnp.float32)
  beta_ref[...] = lax.dot_general(onehot, beta6, (((1,), (0,)), ((), ())),
                                  preferred_element_type=jnp.float32)


def _affine_params(condition_ids, cond_emb, W_hidden, W_gma, W_bta, clngma,
                   clnbta):
  return pl.pallas_call(
      _affine_body,
      out_shape=(jax.ShapeDtypeStruct((B, DIM), jnp.float32),
                 jax.ShapeDtypeStruct((B, DIM), jnp.float32)),
  )(condition_ids.reshape(B, 1).astype(jnp.int32), cond_emb, W_hidden, W_gma,
    W_bta, clngma.reshape(1, DIM), clnbta.reshape(1, DIM))


def _sc_body(ids_hbm, word_hbm, pos_hbm, gamma_hbm, beta_hbm, out_hbm,
             pos_v, w0_v, w1_v, i0_v, i1_v, gam_v, bet_v,
             g0_sem, g1_sem, o0_sem, o1_sem):
  wid = lax.axis_index("s") * NC + lax.axis_index("c")
  s0 = wid * SROWS

  word_bufs = (w0_v, w1_v)
  idx_bufs = (i0_v, i1_v)
  g_sems = (g0_sem, g1_sem)
  o_sems = (o0_sem, o1_sem)

  pltpu.sync_copy(pos_hbm.at[pl.ds(s0, SROWS)], pos_v)
  pltpu.sync_copy(gamma_hbm, gam_v)
  pltpu.sync_copy(beta_hbm, bet_v)

  chunks = [(b, c) for b in range(B) for c in range(NCHUNK)]
  nk = len(chunks)

  def issue_gather(k):
    b, c = chunks[k]
    slot = k % 2
    off = b * S + s0 + c * CHUNK
    pltpu.sync_copy(ids_hbm.at[pl.ds(off, CHUNK)], idx_bufs[slot])
    return pltpu.async_copy(word_hbm.at[idx_bufs[slot]], word_bufs[slot],
                            g_sems[slot])

  gather_descs = {0: issue_gather(0)}
  out_descs = {}

  for k in range(nk):
    b, c = chunks[k]
    slot = k % 2
    if k + 1 < nk:
      if k - 1 >= 0:
        out_descs[k - 1].wait()
      gather_descs[k + 1] = issue_gather(k + 1)
    gather_descs[k].wait()

    wbuf = word_bufs[slot]
    prow = c * CHUNK

    def gbody(g, _, wbuf=wbuf, b=b, prow=prow):
      gs = pl.ds(lax.mul(g, LANES), LANES)
      g16 = gam_v[b, gs]
      b16 = bet_v[b, gs]
      for r in range(CHUNK):
        wbuf[r, gs] = (wbuf[r, gs] + pos_v[prow + r, gs]) * g16 + b16
      return 0

    lax.fori_loop(0, NGROUPS, gbody, 0)

    off = b * S + s0 + c * CHUNK
    out_descs[k] = pltpu.async_copy(wbuf, out_hbm.at[pl.ds(off, CHUNK)],
                                    o_sems[slot])

  out_descs[nk - 2].wait()
  out_descs[nk - 1].wait()


@functools.partial(jax.jit, static_argnames=())
def _sc_gather_affine(ids_flat, word_emb, pos_emb, gamma, beta):
  kern = pl.kernel(
      _sc_body,
      out_type=jax.ShapeDtypeStruct((B * S, DIM), jnp.float32),
      mesh=plsc.VectorSubcoreMesh(core_axis_name="c", subcore_axis_name="s",
                                  num_cores=NC, num_subcores=NS),
      scratch_types=[
          pltpu.VMEM((SROWS, DIM), jnp.float32),
          pltpu.VMEM((CHUNK, DIM), jnp.float32),
          pltpu.VMEM((CHUNK, DIM), jnp.float32),
          pltpu.VMEM((CHUNK,), jnp.int32),
          pltpu.VMEM((CHUNK,), jnp.int32),
          pltpu.VMEM((B, DIM), jnp.float32),
          pltpu.VMEM((B, DIM), jnp.float32),
          pltpu.SemaphoreType.DMA,
          pltpu.SemaphoreType.DMA,
          pltpu.SemaphoreType.DMA,
          pltpu.SemaphoreType.DMA,
      ],
  )
  return kern(ids_flat, word_emb, pos_emb, gamma, beta)


def kernel(input_ids, condition_ids, word_emb, pos_emb, cond_emb, W_hidden,
           W_gma, W_bta, clngma, clnbta):
  gamma, beta = _affine_params(condition_ids, cond_emb, W_hidden, W_gma,
                               W_bta, clngma, clnbta)
  ids_flat = input_ids.reshape(B * S).astype(jnp.int32)
  out = _sc_gather_affine(ids_flat, word_emb, pos_emb.astype(jnp.float32),
                          gamma, beta)
  return out.reshape(B, S, DIM)


# speedup vs baseline: 1.2565x; 1.2565x over previous
import jax
import jax.numpy as jnp
from jax import lax
from jax.experimental import pallas as pl
from jax.experimental.pallas import tpu as pltpu
from jax.experimental.pallas import tpu_sc as plsc

B = 4
S = 2048
DIM = 768
COND_SIZE = 6

NC = 2
NS = 16
NW = NC * NS
SROWS = S // NW
CHUNK = 8
NCHUNK = SROWS // CHUNK
LANES = 16
NGROUPS = DIM // LANES


def _affine_body(cid_ref, cond_emb_ref, wh_ref, wg_ref, wb_ref, clg_ref,
                 clb_ref, gamma_ref, beta_ref):
  conds6 = lax.dot_general(cond_emb_ref[...], wh_ref[...],
                           (((1,), (1,)), ((), ())),
                           preferred_element_type=jnp.float32)
  gamma6 = clg_ref[...] + lax.dot_general(conds6, wg_ref[...],
                                          (((1,), (1,)), ((), ())),
                                          preferred_element_type=jnp.float32)
  beta6 = clb_ref[...] + lax.dot_general(conds6, wb_ref[...],
                                         (((1,), (1,)), ((), ())),
                                         preferred_element_type=jnp.float32)
  onehot = (cid_ref[...] == lax.broadcasted_iota(jnp.int32, (B, COND_SIZE), 1)
            ).astype(jnp.float32)
  gamma_ref[...] = lax.dot_general(onehot, gamma6, (((1,), (0,)), ((), ())),
                                   preferred_element_type=jnp.float32)
  beta_ref[...] = lax.dot_general(onehot, beta6, (((1,), (0,)), ((), ())),
                                  preferred_element_type=jnp.float32)


def _affine_params(condition_ids, cond_emb, W_hidden, W_gma, W_bta, clngma,
                   clnbta):
  return pl.pallas_call(
      _affine_body,
      out_shape=(jax.ShapeDtypeStruct((B, DIM), jnp.float32),
                 jax.ShapeDtypeStruct((B, DIM), jnp.float32)),
  )(condition_ids.reshape(B, 1).astype(jnp.int32), cond_emb, W_hidden, W_gma,
    W_bta, clngma.reshape(1, DIM), clnbta.reshape(1, DIM))


def _sc_body(ids_hbm, word_hbm, pos_hbm, gamma_hbm, beta_hbm, out_hbm,
             w0_v, w1_v, o0_v, o1_v, p0_v, p1_v, i0_v, i1_v, gam_v, bet_v,
             g0_sem, g1_sem, p0_sem, p1_sem, o0_sem, o1_sem):
  wid = lax.axis_index("s") * NC + lax.axis_index("c")
  s0 = wid * SROWS

  word_bufs = (w0_v, w1_v)
  out_bufs = (o0_v, o1_v)
  pos_bufs = (p0_v, p1_v)
  idx_bufs = (i0_v, i1_v)
  g_sems = (g0_sem, g1_sem)
  p_sems = (p0_sem, p1_sem)
  o_sems = (o0_sem, o1_sem)

  pltpu.sync_copy(gamma_hbm, gam_v)
  pltpu.sync_copy(beta_hbm, bet_v)

  def issue(k):
    slot = k % 2
    for b in range(B):
      pltpu.sync_copy(ids_hbm.at[pl.ds(b * S + s0 + k * CHUNK, CHUNK)],
                      idx_bufs[slot].at[pl.ds(b * CHUNK, CHUNK)])
    gd = pltpu.async_copy(word_hbm.at[idx_bufs[slot]], word_bufs[slot],
                          g_sems[slot])
    pd = pltpu.async_copy(pos_hbm.at[pl.ds(s0 + k * CHUNK, CHUNK)],
                          pos_bufs[slot], p_sems[slot])
    return gd, pd

  descs = {0: issue(0)}
  out_descs = {}

  for k in range(NCHUNK):
    slot = k % 2
    if k + 1 < NCHUNK:
      if k - 1 >= 0:
        for d in out_descs[k - 1]:
          d.wait()
      descs[k + 1] = issue(k + 1)
    gd, pd = descs[k]
    gd.wait()
    pd.wait()

    wbuf = word_bufs[slot]
    obuf = out_bufs[slot]
    pbuf = pos_bufs[slot]

    @plsc.parallel_loop(0, NGROUPS, step=1, unroll=1)
    def gbody(g, wbuf=wbuf, obuf=obuf, pbuf=pbuf):
      gs = pl.ds(lax.mul(g, LANES), LANES)
      gms = [gam_v[b, gs] for b in range(B)]
      bts = [bet_v[b, gs] for b in range(B)]
      for r in range(CHUNK):
        p16 = pbuf[r, gs]
        for b in range(B):
          row = b * CHUNK + r
          obuf[row, gs] = (wbuf[row, gs] + p16) * gms[b] + bts[b]

    out_descs[k] = tuple(
        pltpu.async_copy(obuf.at[pl.ds(b * CHUNK, CHUNK)],
                         out_hbm.at[pl.ds(b * S + s0 + k * CHUNK, CHUNK)],
                         o_sems[slot])
        for b in range(B))

  for k in (NCHUNK - 2, NCHUNK - 1):
    for d in out_descs[k]:
      d.wait()


def _sc_gather_affine(ids_flat, word_emb, pos_emb, gamma, beta):
  kern = pl.kernel(
      _sc_body,
      out_type=jax.ShapeDtypeStruct((B * S, DIM), jnp.float32),
      mesh=plsc.VectorSubcoreMesh(core_axis_name="c", subcore_axis_name="s",
                                  num_cores=NC, num_subcores=NS),
      scratch_types=[
          pltpu.VMEM((B * CHUNK, DIM), jnp.float32),
          pltpu.VMEM((B * CHUNK, DIM), jnp.float32),
          pltpu.VMEM((B * CHUNK, DIM), jnp.float32),
          pltpu.VMEM((B * CHUNK, DIM), jnp.float32),
          pltpu.VMEM((CHUNK, DIM), jnp.float32),
          pltpu.VMEM((CHUNK, DIM), jnp.float32),
          pltpu.VMEM((B * CHUNK,), jnp.int32),
          pltpu.VMEM((B * CHUNK,), jnp.int32),
          pltpu.VMEM((B, DIM), jnp.float32),
          pltpu.VMEM((B, DIM), jnp.float32),
          pltpu.SemaphoreType.DMA,
          pltpu.SemaphoreType.DMA,
          pltpu.SemaphoreType.DMA,
          pltpu.SemaphoreType.DMA,
          pltpu.SemaphoreType.DMA,
          pltpu.SemaphoreType.DMA,
      ],
  )
  return kern(ids_flat, word_emb, pos_emb, gamma, beta)


def kernel(input_ids, condition_ids, word_emb, pos_emb, cond_emb, W_hidden,
           W_gma, W_bta, clngma, clnbta):
  gamma, beta = _affine_params(condition_ids, cond_emb, W_hidden, W_gma,
                               W_bta, clngma, clnbta)
  ids_flat = input_ids.reshape(B * S).astype(jnp.int32)
  out = _sc_gather_affine(ids_flat, word_emb, pos_emb.astype(jnp.float32),
                          gamma, beta)
  return out.reshape(B, S, DIM)


# speedup vs baseline: 1.5675x; 1.2476x over previous
import jax
import jax.numpy as jnp
from jax import lax
from jax.experimental import pallas as pl
from jax.experimental.pallas import tpu as pltpu
from jax.experimental.pallas import tpu_sc as plsc

B = 4
S = 2048
DIM = 768
COND_SIZE = 6

NC = 2
NS = 16
NW = NC * NS
SROWS = S // NW
CHUNK = 16
NCHUNK = SROWS // CHUNK
LANES = 16
NGROUPS = DIM // LANES


def _affine_body(cid_ref, cond_emb_ref, wh_ref, wg_ref, wb_ref, clg_ref,
                 clb_ref, gb_ref):
  conds6 = lax.dot_general(cond_emb_ref[...], wh_ref[...],
                           (((1,), (1,)), ((), ())),
                           preferred_element_type=jnp.float32)
  gamma6 = clg_ref[...] + lax.dot_general(conds6, wg_ref[...],
                                          (((1,), (1,)), ((), ())),
                                          preferred_element_type=jnp.float32)
  beta6 = clb_ref[...] + lax.dot_general(conds6, wb_ref[...],
                                         (((1,), (1,)), ((), ())),
                                         preferred_element_type=jnp.float32)
  onehot = (cid_ref[...] == lax.broadcasted_iota(jnp.int32, (B, COND_SIZE), 1)
            ).astype(jnp.float32)
  gb_ref[0:B, :] = lax.dot_general(onehot, gamma6, (((1,), (0,)), ((), ())),
                                   preferred_element_type=jnp.float32)
  gb_ref[B:2 * B, :] = lax.dot_general(onehot, beta6,
                                       (((1,), (0,)), ((), ())),
                                       preferred_element_type=jnp.float32)


def _affine_params(condition_ids, cond_emb, W_hidden, W_gma, W_bta, clngma,
                   clnbta):
  return pl.pallas_call(
      _affine_body,
      out_shape=jax.ShapeDtypeStruct((2 * B, DIM), jnp.float32),
  )(condition_ids.reshape(B, 1), cond_emb, W_hidden, W_gma,
    W_bta, clngma.reshape(1, DIM), clnbta.reshape(1, DIM))


def _sc_body(ids_hbm, word_hbm, pos_hbm, gb_hbm, out_hbm,
             w0_v, w1_v, pos_v, idx_v, gb_v,
             g0_sem, g1_sem, pp_sem, o0_sem, o1_sem, i_sem):
  wid = lax.axis_index("s") * NC + lax.axis_index("c")
  s0 = wid * SROWS

  word_bufs = (w0_v, w1_v)
  g_sems = (g0_sem, g1_sem)
  o_sems = (o0_sem, o1_sem)

  idx_descs = [
      pltpu.async_copy(ids_hbm.at[b, pl.ds(s0 + c * CHUNK, CHUNK)],
                       idx_v.at[c, pl.ds(b * CHUNK, CHUNK)], i_sem)
      for c in range(NCHUNK) for b in range(B)
  ]
  gb_desc = pltpu.async_copy(gb_hbm, gb_v, pp_sem)
  for d in idx_descs:
    d.wait()
  gb_desc.wait()

  def issue_gather(k):
    return pltpu.async_copy(word_hbm.at[idx_v.at[k]], word_bufs[k % 2],
                            g_sems[k % 2])

  def issue_pos(k):
    return pltpu.async_copy(pos_hbm.at[pl.ds(s0 + k * CHUNK, CHUNK)],
                            pos_v, pp_sem)

  gather_descs = {0: issue_gather(0)}
  pos_descs = {0: issue_pos(0)}
  out_descs = {}

  for k in range(NCHUNK):
    slot = k % 2
    if k + 1 < NCHUNK:
      if k - 1 >= 0:
        for d in out_descs[k - 1]:
          d.wait()
      gather_descs[k + 1] = issue_gather(k + 1)
    gather_descs[k].wait()
    pos_descs[k].wait()

    wbuf = word_bufs[slot]

    @plsc.parallel_loop(0, NGROUPS, step=1, unroll=1)
    def gbody(g, wbuf=wbuf):
      gs = pl.ds(lax.mul(g, LANES), LANES)
      gms = [gb_v[b, gs] for b in range(B)]
      bts = [gb_v[B + b, gs] for b in range(B)]
      for r in range(CHUNK):
        p16 = pos_v[r, gs]
        for b in range(B):
          row = b * CHUNK + r
          wbuf[row, gs] = (wbuf[row, gs] + p16) * gms[b] + bts[b]

    if k + 1 < NCHUNK:
      pos_descs[k + 1] = issue_pos(k + 1)

    out_descs[k] = tuple(
        pltpu.async_copy(wbuf.at[pl.ds(b * CHUNK, CHUNK)],
                         out_hbm.at[b, pl.ds(s0 + k * CHUNK, CHUNK)],
                         o_sems[slot])
        for b in range(B))

  for k in (NCHUNK - 2, NCHUNK - 1):
    for d in out_descs[k]:
      d.wait()


def _sc_gather_affine(ids, word_emb, pos_emb, gb):
  kern = pl.kernel(
      _sc_body,
      out_type=jax.ShapeDtypeStruct((B, S, DIM), jnp.float32),
      mesh=plsc.VectorSubcoreMesh(core_axis_name="c", subcore_axis_name="s",
                                  num_cores=NC, num_subcores=NS),
      scratch_types=[
          pltpu.VMEM((B * CHUNK, DIM), jnp.float32),
          pltpu.VMEM((B * CHUNK, DIM), jnp.float32),
          pltpu.VMEM((CHUNK, DIM), jnp.float32),
          pltpu.VMEM((NCHUNK, B * CHUNK), jnp.int32),
          pltpu.VMEM((2 * B, DIM), jnp.float32),
          pltpu.SemaphoreType.DMA,
          pltpu.SemaphoreType.DMA,
          pltpu.SemaphoreType.DMA,
          pltpu.SemaphoreType.DMA,
          pltpu.SemaphoreType.DMA,
          pltpu.SemaphoreType.DMA,
      ],
  )
  return kern(ids, word_emb, pos_emb, gb)


def kernel(input_ids, condition_ids, word_emb, pos_emb, cond_emb, W_hidden,
           W_gma, W_bta, clngma, clnbta):
  if input_ids.dtype != jnp.int32:
    input_ids = input_ids.astype(jnp.int32)
  if condition_ids.dtype != jnp.int32:
    condition_ids = condition_ids.astype(jnp.int32)
  gb = _affine_params(condition_ids, cond_emb, W_hidden, W_gma, W_bta,
                      clngma, clnbta)
  return _sc_gather_affine(input_ids, word_emb, pos_emb, gb)


# speedup vs baseline: 1.6395x; 1.0459x over previous
import jax
import jax.numpy as jnp
from jax import lax
from jax.experimental import pallas as pl
from jax.experimental.pallas import tpu as pltpu
from jax.experimental.pallas import tpu_sc as plsc

B = 4
S = 2048
DIM = 768
COND_SIZE = 6
COND_DIMS = 128

NC = 2
NS = 16
NW = NC * NS
SROWS = S // NW
CHUNK = 16
NCHUNK = SROWS // CHUNK
LANES = 16
NGROUPS = DIM // LANES


KSTEPS = 6
KBLK = DIM // KSTEPS


def _affine_body(cid_ref, cond_emb_ref, wh_ref, wg_ref, wb_ref, clg_ref,
                 clb_ref, gb_ref, acc_ref):
  k = pl.program_id(0)
  conds_k = lax.dot_general(cond_emb_ref[...], wh_ref[...],
                            (((1,), (1,)), ((), ())),
                            preferred_element_type=jnp.float32)
  part_g = lax.dot_general(conds_k, wg_ref[...], (((1,), (1,)), ((), ())),
                           preferred_element_type=jnp.float32)
  part_b = lax.dot_general(conds_k, wb_ref[...], (((1,), (1,)), ((), ())),
                           preferred_element_type=jnp.float32)

  @pl.when(k == 0)
  def _():
    acc_ref[0:COND_SIZE, :] = part_g
    acc_ref[COND_SIZE:2 * COND_SIZE, :] = part_b

  @pl.when(k > 0)
  def _():
    acc_ref[0:COND_SIZE, :] += part_g
    acc_ref[COND_SIZE:2 * COND_SIZE, :] += part_b

  @pl.when(k == KSTEPS - 1)
  def _():
    onehot = (cid_ref[...][:, None]
              == lax.broadcasted_iota(jnp.int32, (B, COND_SIZE), 1)
              ).astype(jnp.float32)
    clg = clg_ref[...][None, :]
    clb = clb_ref[...][None, :]
    gb_ref[0:B, :] = clg + lax.dot_general(
        onehot, acc_ref[0:COND_SIZE, :], (((1,), (0,)), ((), ())),
        preferred_element_type=jnp.float32)
    gb_ref[B:2 * B, :] = clb + lax.dot_general(
        onehot, acc_ref[COND_SIZE:2 * COND_SIZE, :], (((1,), (0,)), ((), ())),
        preferred_element_type=jnp.float32)


def _affine_params(condition_ids, cond_emb, W_hidden, W_gma, W_bta, clngma,
                   clnbta):
  return pl.pallas_call(
      _affine_body,
      grid=(KSTEPS,),
      in_specs=[
          pl.BlockSpec((B,), lambda k: (0,)),
          pl.BlockSpec((COND_SIZE, COND_DIMS), lambda k: (0, 0)),
          pl.BlockSpec((KBLK, COND_DIMS), lambda k: (k, 0)),
          pl.BlockSpec((DIM, KBLK), lambda k: (0, k)),
          pl.BlockSpec((DIM, KBLK), lambda k: (0, k)),
          pl.BlockSpec((DIM,), lambda k: (0,)),
          pl.BlockSpec((DIM,), lambda k: (0,)),
      ],
      out_specs=pl.BlockSpec((2 * B, DIM), lambda k: (0, 0)),
      out_shape=jax.ShapeDtypeStruct((2 * B, DIM), jnp.float32),
      scratch_shapes=[pltpu.VMEM((2 * COND_SIZE, DIM), jnp.float32)],
  )(condition_ids, cond_emb, W_hidden, W_gma, W_bta, clngma, clnbta)


def _sc_body(ids_hbm, word_hbm, pos_hbm, gb_hbm, out_hbm,
             w0_v, w1_v, pos_v, idx_v, gb_v,
             g0_sem, g1_sem, pp_sem, o0_sem, o1_sem, i_sem):
  wid = lax.axis_index("s") * NC + lax.axis_index("c")
  s0 = wid * SROWS

  word_bufs = (w0_v, w1_v)
  g_sems = (g0_sem, g1_sem)
  o_sems = (o0_sem, o1_sem)

  idx_descs = [
      pltpu.async_copy(ids_hbm.at[b, pl.ds(s0 + c * CHUNK, CHUNK)],
                       idx_v.at[c, pl.ds(b * CHUNK, CHUNK)], i_sem)
      for c in range(NCHUNK) for b in range(B)
  ]
  gb_desc = pltpu.async_copy(gb_hbm, gb_v, pp_sem)
  for d in idx_descs:
    d.wait()
  gb_desc.wait()

  def issue_gather(k):
    return pltpu.async_copy(word_hbm.at[idx_v.at[k]], word_bufs[k % 2],
                            g_sems[k % 2])

  def issue_pos(k):
    return pltpu.async_copy(pos_hbm.at[pl.ds(s0 + k * CHUNK, CHUNK)],
                            pos_v, pp_sem)

  gather_descs = {0: issue_gather(0)}
  pos_descs = {0: issue_pos(0)}
  out_descs = {}

  for k in range(NCHUNK):
    slot = k % 2
    if k + 1 < NCHUNK:
      if k - 1 >= 0:
        for d in out_descs[k - 1]:
          d.wait()
      gather_descs[k + 1] = issue_gather(k + 1)
    gather_descs[k].wait()
    pos_descs[k].wait()

    wbuf = word_bufs[slot]

    @plsc.parallel_loop(0, NGROUPS, step=1, unroll=1)
    def gbody(g, wbuf=wbuf):
      gs = pl.ds(lax.mul(g, LANES), LANES)
      gms = [gb_v[b, gs] for b in range(B)]
      bts = [gb_v[B + b, gs] for b in range(B)]
      for r in range(CHUNK):
        p16 = pos_v[r, gs]
        for b in range(B):
          row = b * CHUNK + r
          wbuf[row, gs] = (wbuf[row, gs] + p16) * gms[b] + bts[b]

    if k + 1 < NCHUNK:
      pos_descs[k + 1] = issue_pos(k + 1)

    out_descs[k] = tuple(
        pltpu.async_copy(wbuf.at[pl.ds(b * CHUNK, CHUNK)],
                         out_hbm.at[b, pl.ds(s0 + k * CHUNK, CHUNK)],
                         o_sems[slot])
        for b in range(B))

  for k in (NCHUNK - 2, NCHUNK - 1):
    for d in out_descs[k]:
      d.wait()


def _sc_gather_affine(ids, word_emb, pos_emb, gb):
  kern = pl.kernel(
      _sc_body,
      out_type=jax.ShapeDtypeStruct((B, S, DIM), jnp.float32),
      mesh=plsc.VectorSubcoreMesh(core_axis_name="c", subcore_axis_name="s",
                                  num_cores=NC, num_subcores=NS),
      scratch_types=[
          pltpu.VMEM((B * CHUNK, DIM), jnp.float32),
          pltpu.VMEM((B * CHUNK, DIM), jnp.float32),
          pltpu.VMEM((CHUNK, DIM), jnp.float32),
          pltpu.VMEM((NCHUNK, B * CHUNK), jnp.int32),
          pltpu.VMEM((2 * B, DIM), jnp.float32),
          pltpu.SemaphoreType.DMA,
          pltpu.SemaphoreType.DMA,
          pltpu.SemaphoreType.DMA,
          pltpu.SemaphoreType.DMA,
          pltpu.SemaphoreType.DMA,
          pltpu.SemaphoreType.DMA,
      ],
  )
  return kern(ids, word_emb, pos_emb, gb)


def kernel(input_ids, condition_ids, word_emb, pos_emb, cond_emb, W_hidden,
           W_gma, W_bta, clngma, clnbta):
  if input_ids.dtype != jnp.int32:
    input_ids = input_ids.astype(jnp.int32)
  if condition_ids.dtype != jnp.int32:
    condition_ids = condition_ids.astype(jnp.int32)
  gb = _affine_params(condition_ids, cond_emb, W_hidden, W_gma, W_bta,
                      clngma, clnbta)
  return _sc_gather_affine(input_ids, word_emb, pos_emb, gb)
